# Initial kernel scaffold; baseline (speedup 1.0000x reference)
import functools

import jax
import jax.numpy as jnp
from jax import lax
from jax.experimental import pallas as pl
from jax.experimental.pallas import tpu as pltpu
from jax.experimental.pallas import tpu_sc as plsc

B, N, V = 8, 4096, 1024
NC, NS, L = 2, 16, 16
IDX_ROWS = N // 128


def _body(flat_hbm, idvec_hbm, out_hbm, idvec_v, idx_v, vals_v, out_v, sem):
    wid = lax.axis_index("s") * NC + lax.axis_index("c")

    @pl.when(wid < B)
    def _():
        b = wid
        pltpu.sync_copy(idvec_hbm, idvec_v)
        id_vec = idvec_v[...]
        lane = lax.broadcasted_iota(jnp.int32, (L,), 0)
        zero = jnp.zeros((L,), jnp.int32)
        base = b * (N * V)

        def build(j, carry):
            for l in range(8):
                i = j * 128 + l * L + lane
                idx_v[j, pl.ds(l * L, L)] = base + i * V + id_vec
                out_v[pl.ds(j * 128 + l * L, L)] = zero
            return carry

        lax.fori_loop(0, IDX_ROWS, build, 0)

        for lo in range(0, IDX_ROWS, 16):
            copies = [
                pltpu.async_copy(flat_hbm.at[idx_v.at[j]], vals_v.at[j], sem)
                for j in range(lo, lo + 16)
            ]
            for cp in copies:
                cp.wait()

        def compact(j, cnt):
            for l in range(8):
                v = vals_v[j, pl.ds(l * L, L)]
                m = v != 0.0
                csum = plsc.cumsum(m.astype(jnp.int32))
                pos = cnt + csum - 1
                gidx = j * 128 + l * L + lane
                plsc.store_scatter(out_v, [pos], gidx, mask=m)
                cnt = cnt + plsc.all_reduce_population_count(m)
            return cnt

        lax.fori_loop(0, IDX_ROWS, compact, jnp.zeros((L,), jnp.int32))
        pltpu.sync_copy(out_v, out_hbm.at[b])


@functools.partial(
    pl.kernel,
    out_type=jax.ShapeDtypeStruct((B, N), jnp.int32),
    mesh=plsc.VectorSubcoreMesh(core_axis_name="c", subcore_axis_name="s"),
    scratch_types=[
        pltpu.VMEM((L,), jnp.int32),
        pltpu.VMEM((IDX_ROWS, 128), jnp.int32),
        pltpu.VMEM((IDX_ROWS, 128), jnp.float32),
        pltpu.VMEM((N,), jnp.int32),
        pltpu.SemaphoreType.DMA,
    ],
)
def _filter_sc(flat_hbm, idvec_hbm, out_hbm, idvec_v, idx_v, vals_v, out_v,
               sem):
    _body(flat_hbm, idvec_hbm, out_hbm, idvec_v, idx_v, vals_v, out_v, sem)


def kernel(one_hot, id):
    flat = one_hot.reshape(-1)
    idvec = jnp.full((L,), id, dtype=jnp.int32)
    return _filter_sc(flat, idvec)


# speedup vs baseline: 3.1601x; 3.1601x over previous
import functools

import jax
import jax.numpy as jnp
from jax import lax
from jax.experimental import pallas as pl
from jax.experimental.pallas import tpu as pltpu
from jax.experimental.pallas import tpu_sc as plsc

B, N, V = 8, 4096, 1024
NC, NS, L = 2, 16, 16
IDX_ROWS = N // 128


def _body(flat_hbm, idvec_hbm, out_hbm, idvec_v, idx_v, vals_v, out_v, sem):
    wid = lax.axis_index("s") * NC + lax.axis_index("c")

    @pl.when(wid < B)
    def _():
        b = wid
        pltpu.sync_copy(idvec_hbm, idvec_v)
        id_vec = idvec_v[...]
        lane = lax.broadcasted_iota(jnp.int32, (L,), 0)
        zero = jnp.zeros((L,), jnp.int32)
        base = b * (N * V)

        def build(j, carry):
            for l in range(8):
                i = j * 128 + l * L + lane
                idx_v[j, pl.ds(l * L, L)] = base + i * V + id_vec
                out_v[pl.ds(j * 128 + l * L, L)] = zero
            return carry

        lax.fori_loop(0, IDX_ROWS, build, 0)

        for lo in range(0, IDX_ROWS, 16):
            copies = [
                pltpu.async_copy(flat_hbm.at[idx_v.at[j]], vals_v.at[j], sem)
                for j in range(lo, lo + 16)
            ]
            for cp in copies:
                cp.wait()

        def compact(j, cnt):
            for l in range(8):
                v = vals_v[j, pl.ds(l * L, L)]
                m = v != 0.0
                csum = plsc.cumsum(m.astype(jnp.int32))
                pos = cnt + csum - 1
                gidx = j * 128 + l * L + lane
                plsc.store_scatter(out_v, [pos], gidx, mask=m)
                cnt = cnt + plsc.all_reduce_population_count(m)
            return cnt

        lax.fori_loop(0, IDX_ROWS, compact, jnp.zeros((L,), jnp.int32))
        pltpu.sync_copy(out_v, out_hbm.at[b])


@functools.partial(
    pl.kernel,
    out_type=jax.ShapeDtypeStruct((B, N), jnp.int32),
    mesh=plsc.VectorSubcoreMesh(core_axis_name="c", subcore_axis_name="s"),
    scratch_types=[
        pltpu.VMEM((L,), jnp.int32),
        pltpu.VMEM((IDX_ROWS, 128), jnp.int32),
        pltpu.VMEM((IDX_ROWS, 128), jnp.float32),
        pltpu.VMEM((N,), jnp.int32),
        pltpu.SemaphoreType.DMA,
    ],
    compiler_params=pltpu.CompilerParams(needs_layout_passes=False),
)
def _filter_sc(flat_hbm, idvec_hbm, out_hbm, idvec_v, idx_v, vals_v, out_v,
               sem):
    _body(flat_hbm, idvec_hbm, out_hbm, idvec_v, idx_v, vals_v, out_v, sem)


def kernel(one_hot, id):
    flat = one_hot.reshape(-1)
    idvec = jnp.full((L,), id, dtype=jnp.int32)
    return _filter_sc(flat, idvec)


# speedup vs baseline: 8.1700x; 2.5854x over previous
import functools

import jax
import jax.numpy as jnp
from jax import lax
from jax.experimental import pallas as pl
from jax.experimental.pallas import tpu as pltpu
from jax.experimental.pallas import tpu_sc as plsc

B, N, V = 8, 4096, 1024
NC, NS, L = 2, 16, 16
CH = 256
NCHUNK = N // CH


def _body(rows_hbm, idvec_hbm, out_hbm, idvec_v, vals0, vals1, out_v,
          sem0, sem1):
    wid = lax.axis_index("s") * NC + lax.axis_index("c")

    @pl.when(wid < B)
    def _():
        b = wid
        pltpu.sync_copy(idvec_hbm, idvec_v)
        idv = idvec_v[...]
        id_lane = idv & 127
        col0 = pl.multiple_of((idv[0] >> 7) << 7, 128)
        lane = lax.broadcasted_iota(jnp.int32, (L,), 0)
        zero = jnp.zeros((L,), jnp.int32)
        row_base = b * N

        def src(k):
            return rows_hbm.at[pl.ds(row_base + k * CH, CH), pl.ds(col0, 128)]

        pltpu.async_copy(src(0), vals0, sem0)
        pltpu.async_copy(src(1), vals1, sem1)

        def zfill(j, carry):
            out_v[pl.ds(j * L, L)] = zero
            return carry

        lax.fori_loop(0, N // L, zfill, 0)

        def compact_chunk(vals_ref, k, cnt):
            def step(t, cnt):
                r = t * L + lane
                v = plsc.load_gather(vals_ref, [r, id_lane])
                m = v != 0.0
                csum = plsc.cumsum(m.astype(jnp.int32))
                pos = cnt + csum - 1
                plsc.store_scatter(out_v, [pos], k * CH + r, mask=m)
                return cnt + plsc.all_reduce_population_count(m)

            return lax.fori_loop(0, CH // L, step, cnt)

        def outer(g, cnt):
            k0 = 2 * g
            pltpu.make_async_copy(src(k0), vals0, sem0).wait()
            cnt = compact_chunk(vals0, k0, cnt)

            @pl.when(k0 + 2 < NCHUNK)
            def _():
                pltpu.async_copy(src(k0 + 2), vals0, sem0)

            k1 = 2 * g + 1
            pltpu.make_async_copy(src(k1), vals1, sem1).wait()
            cnt = compact_chunk(vals1, k1, cnt)

            @pl.when(k1 + 2 < NCHUNK)
            def _():
                pltpu.async_copy(src(k1 + 2), vals1, sem1)

            return cnt

        lax.fori_loop(0, NCHUNK // 2, outer, jnp.zeros((L,), jnp.int32))
        pltpu.sync_copy(out_v, out_hbm.at[b])


@functools.partial(
    pl.kernel,
    out_type=jax.ShapeDtypeStruct((B, N), jnp.int32),
    mesh=plsc.VectorSubcoreMesh(core_axis_name="c", subcore_axis_name="s"),
    scratch_types=[
        pltpu.VMEM((L,), jnp.int32),
        pltpu.VMEM((CH, 128), jnp.float32),
        pltpu.VMEM((CH, 128), jnp.float32),
        pltpu.VMEM((N,), jnp.int32),
        pltpu.SemaphoreType.DMA,
        pltpu.SemaphoreType.DMA,
    ],
    compiler_params=pltpu.CompilerParams(needs_layout_passes=False),
)
def _filter_sc(rows_hbm, idvec_hbm, out_hbm, idvec_v, vals0, vals1, out_v,
               sem0, sem1):
    _body(rows_hbm, idvec_hbm, out_hbm, idvec_v, vals0, vals1, out_v,
          sem0, sem1)


def kernel(one_hot, id):
    rows = one_hot.reshape(B * N, V)
    idvec = jnp.full((L,), id, dtype=jnp.int32)
    return _filter_sc(rows, idvec)


# speedup vs baseline: 13.0228x; 1.5940x over previous
import functools

import jax
import jax.numpy as jnp
from jax import lax
from jax.experimental import pallas as pl
from jax.experimental.pallas import tpu as pltpu
from jax.experimental.pallas import tpu_sc as plsc

B, N, V = 8, 4096, 1024
NC, NS, L = 2, 16, 16
WPB = 4
Q = N // WPB
CH = 256
NCHUNK = Q // CH


def _body(rows_hbm, idvec_hbm, out_hbm, idvec_v, vals0, vals1, loc_v, idx_v,
          acc_sm, out_sh, sem0, sem1, sem2):
    c = lax.axis_index("c")
    s = lax.axis_index("s")
    batch = c * (NS // WPB) + (s >> 2)
    bloc = s >> 2
    q = s & (WPB - 1)
    row_base = batch * N + q * Q

    acc_sm[0] = 0

    pltpu.sync_copy(idvec_hbm, idvec_v)
    idv = idvec_v[...]
    id_lane = idv & 127
    col0 = pl.multiple_of((idv[0] >> 7) << 7, 128)
    lane = lax.broadcasted_iota(jnp.int32, (L,), 0)
    zero = jnp.zeros((L,), jnp.int32)

    def src(k):
        return rows_hbm.at[pl.ds(row_base + k * CH, CH), pl.ds(col0, 128)]

    pltpu.async_copy(src(0), vals0, sem0)
    pltpu.async_copy(src(1), vals1, sem1)

    def zfill(j, carry):
        loc_v[pl.ds(j * L, L)] = zero
        return carry

    lax.fori_loop(0, Q // L, zfill, 0)

    def compact_chunk(vals_ref, k, cnt):
        def step(t, cnt):
            r = t * L + lane
            v = plsc.load_gather(vals_ref, [r, id_lane])
            m = v != 0.0
            csum = plsc.cumsum(m.astype(jnp.int32))
            pos = cnt + csum - 1
            plsc.store_scatter(loc_v, [pos], q * Q + k * CH + r, mask=m)
            return cnt + plsc.all_reduce_population_count(m)

        return lax.fori_loop(0, CH // L, step, cnt)

    def outer(g, cnt):
        k0 = 2 * g
        pltpu.make_async_copy(src(k0), vals0, sem0).wait()
        cnt = compact_chunk(vals0, k0, cnt)

        @pl.when(k0 + 2 < NCHUNK)
        def _():
            pltpu.async_copy(src(k0 + 2), vals0, sem0)

        k1 = 2 * g + 1
        pltpu.make_async_copy(src(k1), vals1, sem1).wait()
        cnt = compact_chunk(vals1, k1, cnt)

        @pl.when(k1 + 2 < NCHUNK)
        def _():
            pltpu.async_copy(src(k1 + 2), vals1, sem1)

        return cnt

    cnt = lax.fori_loop(0, NCHUNK // 2, outer, jnp.zeros((L,), jnp.int32))
    cnt_s = cnt[0]

    init = jnp.where(q == 0, 1, 0)

    def poll_cond(carry):
        v, i = carry
        return (v == 0) & (i < (1 << 20))

    def poll_body(carry):
        v, i = carry
        return plsc.fetch_and_add(acc_sm, 0, subcore_id=s), i + 1

    v, _ = lax.while_loop(poll_cond, poll_body, (init, jnp.int32(0)))
    prefix_s = v - 1

    @pl.when(q < WPB - 1)
    def _():
        plsc.fetch_and_add(acc_sm, prefix_s + cnt_s + 1, subcore_id=s + 1)

    spares = q * Q - prefix_s
    front0 = bloc * N + prefix_s
    tail0 = bloc * N + (N - 1) - spares + cnt

    def ifill(t, carry):
        jvec = t * L + lane
        idx_v[pl.ds(t * L, L)] = jnp.where(
            jvec < cnt, front0 + jvec, tail0 - jvec
        )
        return carry

    lax.fori_loop(0, Q // L, ifill, 0)
    pltpu.async_copy(loc_v, out_sh.at[idx_v], sem2).wait()
    plsc.subcore_barrier()

    @pl.when(q == 0)
    def _():
        pltpu.sync_copy(out_sh.at[pl.ds(bloc * N, N)], out_hbm.at[batch])


@functools.partial(
    pl.kernel,
    out_type=jax.ShapeDtypeStruct((B, N), jnp.int32),
    mesh=plsc.VectorSubcoreMesh(core_axis_name="c", subcore_axis_name="s"),
    scratch_types=[
        pltpu.VMEM((L,), jnp.int32),
        pltpu.VMEM((CH, 128), jnp.float32),
        pltpu.VMEM((CH, 128), jnp.float32),
        pltpu.VMEM((Q,), jnp.int32),
        pltpu.VMEM((Q,), jnp.int32),
        pltpu.SMEM((1,), jnp.int32),
        pltpu.VMEM_SHARED((N * NS // WPB,), jnp.int32),
        pltpu.SemaphoreType.DMA,
        pltpu.SemaphoreType.DMA,
        pltpu.SemaphoreType.DMA,
    ],
    compiler_params=pltpu.CompilerParams(needs_layout_passes=False),
)
def _filter_sc(rows_hbm, idvec_hbm, out_hbm, idvec_v, vals0, vals1, loc_v,
               idx_v, acc_sm, out_sh, sem0, sem1, sem2):
    _body(rows_hbm, idvec_hbm, out_hbm, idvec_v, vals0, vals1, loc_v, idx_v,
          acc_sm, out_sh, sem0, sem1, sem2)


def kernel(one_hot, id):
    rows = one_hot.reshape(B * N, V)
    idvec = jnp.full((L,), id, dtype=jnp.int32)
    return _filter_sc(rows, idvec)
